# baseline (speedup 1.0000x reference)
import functools

import jax
import jax.numpy as jnp
import numpy as np
from jax.experimental import pallas as pl
from jax.experimental.pallas import tpu as pltpu

B = 2048
D_IN = 128
D_OUT = 128
K = 16
BT = 1024

_KNOTS = np.linspace(0.0, 1.0, K, dtype=np.float32)
_HS = [float(_KNOTS[i + 1] - _KNOTS[i]) for i in range(K - 1)]
_X0 = float(_KNOTS[0])
_XN = float(_KNOTS[-1])
_DX = float(np.float32(_XN - _X0) / (K - 1))
_INV_DX = float(np.float32(1.0) / np.float32(np.float32(_DX) + 1e-12))


def _compute_slopes(cs):
    ms = [None] * K
    deltas = []
    for k in range(K - 1):
        deltas.append((cs[k + 1] - cs[k]) / (_HS[k] + 1e-12))
    for k in range(1, K - 1):
        h0 = _HS[k - 1]
        h1 = _HS[k]
        w1 = 2.0 * h1 + h0
        w2 = h1 + 2.0 * h0
        del0 = deltas[k - 1]
        del1 = deltas[k]
        same_sign = del0 * del1 > 0
        denom = w1 / (del0 + 1e-12) + w2 / (del1 + 1e-12)
        d_int = (w1 + w2) / (denom + 1e-12)
        ms[k] = jnp.where(same_sign, d_int, jnp.zeros_like(d_int))
    d0 = ((2.0 * _HS[0] + _HS[1]) * deltas[0] - _HS[0] * deltas[1]) / (
        _HS[0] + _HS[1] + 1e-12)
    d0 = jnp.where(d0 * deltas[0] <= 0, jnp.zeros_like(d0), d0)
    d0 = jnp.where(
        (deltas[0] * deltas[1] < 0) & (jnp.abs(d0) > 3 * jnp.abs(deltas[0])),
        3 * deltas[0], d0)
    ms[0] = d0
    dn = ((2.0 * _HS[-1] + _HS[-2]) * deltas[-1] - _HS[-1] * deltas[-2]) / (
        _HS[-1] + _HS[-2] + 1e-12)
    dn = jnp.where(dn * deltas[-1] <= 0, jnp.zeros_like(dn), dn)
    dn = jnp.where(
        (deltas[-1] * deltas[-2] < 0) & (jnp.abs(dn) > 3 * jnp.abs(deltas[-1])),
        3 * deltas[-1], dn)
    ms[K - 1] = dn
    return ms


def _body(x_ref, c_ref, bias_ref, o_ref, cm_ref):
    @pl.when(pl.program_id(0) == 0)
    def _():
        cs = [c_ref[k].astype(jnp.float32) for k in range(K)]
        ms = _compute_slopes(cs)
        for k in range(K):
            cm_ref[2 * D_IN * k:2 * D_IN * k + D_IN] = c_ref[k]
            cm_ref[2 * D_IN * k + D_IN:2 * D_IN * (k + 1)] = (
                ms[k].astype(jnp.bfloat16))

    x = x_ref[...]
    u = (x - _X0) * _INV_DX
    idxf = jnp.clip(jnp.floor(u), 0.0, K - 2.0)
    t = (u - idxf).astype(jnp.bfloat16)
    idxb = idxf.astype(jnp.bfloat16)
    t2 = t * t
    t3 = t2 * t
    h00 = 2 * t3 - 3 * t2 + 1
    h10 = (t3 - 2 * t2 + t) * jnp.bfloat16(_DX)
    h01 = -2 * t3 + 3 * t2
    h11 = (t3 - t2) * jnp.bfloat16(_DX)
    one = jnp.ones_like(t)
    zero = jnp.zeros_like(t)
    m_prev = None
    ws = []
    for j in range(K):
        m_j = (zero if j == K - 1 else
               jnp.where(idxb == jnp.bfloat16(j), one, zero))
        if j == 0:
            wc = m_j * h00
            wm = m_j * h10
        elif j == K - 1:
            wc = m_prev * h01
            wm = m_prev * h11
        else:
            wc = m_j * h00 + m_prev * h01
            wm = m_j * h10 + m_prev * h11
        m_prev = m_j
        ws.append(wc)
        ws.append(wm)
    w = jnp.concatenate(ws, axis=1)
    o_ref[...] = bias_ref[...] + jax.lax.dot_general(
        w, cm_ref[...], (((1,), (0,)), ((), ())),
        preferred_element_type=jnp.float32)


@functools.partial(jax.jit, static_argnames=("interpret",))
def kernel(x, knots, coeffs, bias, interpret=False):
    del knots
    c_t = jnp.transpose(coeffs.astype(jnp.bfloat16), (2, 1, 0))
    bias2 = bias.reshape(1, D_OUT)
    out = pl.pallas_call(
        _body,
        grid=(B // BT,),
        in_specs=[
            pl.BlockSpec((BT, D_IN), lambda b: (b, 0)),
            pl.BlockSpec((K, D_IN, D_OUT), lambda b: (0, 0, 0)),
            pl.BlockSpec((1, D_OUT), lambda b: (0, 0)),
        ],
        out_specs=pl.BlockSpec((BT, D_OUT), lambda b: (b, 0)),
        out_shape=jax.ShapeDtypeStruct((B, D_OUT), jnp.float32),
        scratch_shapes=[
            pltpu.VMEM((2 * K * D_IN, D_OUT), jnp.bfloat16),
        ],
        interpret=interpret,
    )(x, c_t, bias2)
    return out


# speedup vs baseline: 1.0168x; 1.0168x over previous
import functools

import jax
import jax.numpy as jnp
import numpy as np
from jax.experimental import pallas as pl
from jax.experimental.pallas import tpu as pltpu

B = 2048
D_IN = 128
D_OUT = 128
K = 16
BT = 2048

_KNOTS = np.linspace(0.0, 1.0, K, dtype=np.float32)
_HS = [float(_KNOTS[i + 1] - _KNOTS[i]) for i in range(K - 1)]
_X0 = float(_KNOTS[0])
_XN = float(_KNOTS[-1])
_DX = float(np.float32(_XN - _X0) / (K - 1))
_INV_DX = float(np.float32(1.0) / np.float32(np.float32(_DX) + 1e-12))


def _compute_slopes(cs):
    ms = [None] * K
    deltas = []
    for k in range(K - 1):
        deltas.append((cs[k + 1] - cs[k]) / (_HS[k] + 1e-12))
    for k in range(1, K - 1):
        h0 = _HS[k - 1]
        h1 = _HS[k]
        w1 = 2.0 * h1 + h0
        w2 = h1 + 2.0 * h0
        del0 = deltas[k - 1]
        del1 = deltas[k]
        same_sign = del0 * del1 > 0
        denom = w1 / (del0 + 1e-12) + w2 / (del1 + 1e-12)
        d_int = (w1 + w2) / (denom + 1e-12)
        ms[k] = jnp.where(same_sign, d_int, jnp.zeros_like(d_int))
    d0 = ((2.0 * _HS[0] + _HS[1]) * deltas[0] - _HS[0] * deltas[1]) / (
        _HS[0] + _HS[1] + 1e-12)
    d0 = jnp.where(d0 * deltas[0] <= 0, jnp.zeros_like(d0), d0)
    d0 = jnp.where(
        (deltas[0] * deltas[1] < 0) & (jnp.abs(d0) > 3 * jnp.abs(deltas[0])),
        3 * deltas[0], d0)
    ms[0] = d0
    dn = ((2.0 * _HS[-1] + _HS[-2]) * deltas[-1] - _HS[-1] * deltas[-2]) / (
        _HS[-1] + _HS[-2] + 1e-12)
    dn = jnp.where(dn * deltas[-1] <= 0, jnp.zeros_like(dn), dn)
    dn = jnp.where(
        (deltas[-1] * deltas[-2] < 0) & (jnp.abs(dn) > 3 * jnp.abs(deltas[-1])),
        3 * deltas[-1], dn)
    ms[K - 1] = dn
    return ms


def _body(x_ref, c_ref, bias_ref, o_ref, cm_ref):
    @pl.when(pl.program_id(0) == 0)
    def _():
        cs = [c_ref[k].astype(jnp.float32) for k in range(K)]
        ms = _compute_slopes(cs)
        for k in range(K):
            cm_ref[2 * D_IN * k:2 * D_IN * k + D_IN] = c_ref[k]
            cm_ref[2 * D_IN * k + D_IN:2 * D_IN * (k + 1)] = (
                ms[k].astype(jnp.bfloat16))

    x = x_ref[...]
    u = (x - _X0) * _INV_DX
    idxf = jnp.clip(jnp.floor(u), 0.0, K - 2.0)
    t = (u - idxf).astype(jnp.bfloat16)
    idxb = idxf.astype(jnp.bfloat16)
    t2 = t * t
    t3 = t2 * t
    h00 = 2 * t3 - 3 * t2 + 1
    h10 = (t3 - 2 * t2 + t) * jnp.bfloat16(_DX)
    h01 = -2 * t3 + 3 * t2
    h11 = (t3 - t2) * jnp.bfloat16(_DX)
    one = jnp.ones_like(t)
    zero = jnp.zeros_like(t)
    m_prev = None
    ws = []
    for j in range(K):
        m_j = (zero if j == K - 1 else
               jnp.where(idxb == jnp.bfloat16(j), one, zero))
        if j == 0:
            wc = m_j * h00
            wm = m_j * h10
        elif j == K - 1:
            wc = m_prev * h01
            wm = m_prev * h11
        else:
            wc = m_j * h00 + m_prev * h01
            wm = m_j * h10 + m_prev * h11
        m_prev = m_j
        ws.append(wc)
        ws.append(wm)
    w = jnp.concatenate(ws, axis=1)
    o_ref[...] = bias_ref[...] + jax.lax.dot_general(
        w, cm_ref[...], (((1,), (0,)), ((), ())),
        preferred_element_type=jnp.float32)


@functools.partial(jax.jit, static_argnames=("interpret",))
def kernel(x, knots, coeffs, bias, interpret=False):
    del knots
    c_t = jnp.transpose(coeffs.astype(jnp.bfloat16), (2, 1, 0))
    bias2 = bias.reshape(1, D_OUT)
    out = pl.pallas_call(
        _body,
        grid=(B // BT,),
        in_specs=[
            pl.BlockSpec((BT, D_IN), lambda b: (b, 0)),
            pl.BlockSpec((K, D_IN, D_OUT), lambda b: (0, 0, 0)),
            pl.BlockSpec((1, D_OUT), lambda b: (0, 0)),
        ],
        out_specs=pl.BlockSpec((BT, D_OUT), lambda b: (b, 0)),
        out_shape=jax.ShapeDtypeStruct((B, D_OUT), jnp.float32),
        scratch_shapes=[
            pltpu.VMEM((2 * K * D_IN, D_OUT), jnp.bfloat16),
        ],
        interpret=interpret,
    )(x, c_t, bias2)
    return out


# speedup vs baseline: 1.0172x; 1.0004x over previous
import jax
import jax.numpy as jnp
import numpy as np
from jax.experimental import pallas as pl
from jax.experimental.pallas import tpu as pltpu

B = 2048
D_IN = 128
D_OUT = 128
K = 16
BT = 2048

_KNOTS = np.linspace(0.0, 1.0, K, dtype=np.float32)
_HS = [float(_KNOTS[i + 1] - _KNOTS[i]) for i in range(K - 1)]
_X0 = float(_KNOTS[0])
_XN = float(_KNOTS[-1])
_DX = float(np.float32(_XN - _X0) / (K - 1))
_INV_DX = float(np.float32(1.0) / np.float32(np.float32(_DX) + 1e-12))


def _compute_slopes(cs):
    ms = [None] * K
    deltas = []
    for k in range(K - 1):
        deltas.append((cs[k + 1] - cs[k]) / (_HS[k] + 1e-12))
    for k in range(1, K - 1):
        h0 = _HS[k - 1]
        h1 = _HS[k]
        w1 = 2.0 * h1 + h0
        w2 = h1 + 2.0 * h0
        del0 = deltas[k - 1]
        del1 = deltas[k]
        same_sign = del0 * del1 > 0
        denom = w1 / (del0 + 1e-12) + w2 / (del1 + 1e-12)
        d_int = (w1 + w2) / (denom + 1e-12)
        ms[k] = jnp.where(same_sign, d_int, jnp.zeros_like(d_int))
    d0 = ((2.0 * _HS[0] + _HS[1]) * deltas[0] - _HS[0] * deltas[1]) / (
        _HS[0] + _HS[1] + 1e-12)
    d0 = jnp.where(d0 * deltas[0] <= 0, jnp.zeros_like(d0), d0)
    d0 = jnp.where(
        (deltas[0] * deltas[1] < 0) & (jnp.abs(d0) > 3 * jnp.abs(deltas[0])),
        3 * deltas[0], d0)
    ms[0] = d0
    dn = ((2.0 * _HS[-1] + _HS[-2]) * deltas[-1] - _HS[-1] * deltas[-2]) / (
        _HS[-1] + _HS[-2] + 1e-12)
    dn = jnp.where(dn * deltas[-1] <= 0, jnp.zeros_like(dn), dn)
    dn = jnp.where(
        (deltas[-1] * deltas[-2] < 0) & (jnp.abs(dn) > 3 * jnp.abs(deltas[-1])),
        3 * deltas[-1], dn)
    ms[K - 1] = dn
    return ms


def _body(x_ref, c_ref, bias_ref, o_ref, cm_ref):
    @pl.when(pl.program_id(0) == 0)
    def _():
        cs = [c_ref[k].astype(jnp.float32) for k in range(K)]
        ms = _compute_slopes(cs)
        for k in range(K):
            cm_ref[2 * D_IN * k:2 * D_IN * k + D_IN] = c_ref[k]
            cm_ref[2 * D_IN * k + D_IN:2 * D_IN * (k + 1)] = (
                ms[k].astype(jnp.bfloat16))

    x = x_ref[...]
    u = (x - _X0) * _INV_DX
    idxf = jnp.clip(jnp.floor(u), 0.0, K - 2.0)
    t = (u - idxf).astype(jnp.bfloat16)
    idxb = idxf.astype(jnp.bfloat16)
    t2 = t * t
    t3 = t2 * t
    h00 = 2 * t3 - 3 * t2 + 1
    h10 = (t3 - 2 * t2 + t) * jnp.bfloat16(_DX)
    h01 = -2 * t3 + 3 * t2
    h11 = (t3 - t2) * jnp.bfloat16(_DX)
    one = jnp.ones_like(t)
    zero = jnp.zeros_like(t)
    m_prev = None
    ws = []
    for j in range(K):
        m_j = (zero if j == K - 1 else
               jnp.where(idxb == jnp.bfloat16(j), one, zero))
        if j == 0:
            wc = m_j * h00
            wm = m_j * h10
        elif j == K - 1:
            wc = m_prev * h01
            wm = m_prev * h11
        else:
            wc = m_j * h00 + m_prev * h01
            wm = m_j * h10 + m_prev * h11
        m_prev = m_j
        ws.append(wc)
        ws.append(wm)
    w = jnp.concatenate(ws, axis=1)
    o_ref[...] = bias_ref[...] + jax.lax.dot_general(
        w, cm_ref[...], (((1,), (0,)), ((), ())),
        preferred_element_type=jnp.float32)


@jax.jit
def kernel(x, knots, coeffs, bias):
    del knots
    c_t = jnp.transpose(coeffs.astype(jnp.bfloat16), (2, 1, 0))
    bias2 = bias.reshape(1, D_OUT)
    out = pl.pallas_call(
        _body,
        grid=(B // BT,),
        in_specs=[
            pl.BlockSpec((BT, D_IN), lambda b: (b, 0)),
            pl.BlockSpec((K, D_IN, D_OUT), lambda b: (0, 0, 0)),
            pl.BlockSpec((1, D_OUT), lambda b: (0, 0)),
        ],
        out_specs=pl.BlockSpec((BT, D_OUT), lambda b: (b, 0)),
        out_shape=jax.ShapeDtypeStruct((B, D_OUT), jnp.float32),
        scratch_shapes=[
            pltpu.VMEM((2 * K * D_IN, D_OUT), jnp.bfloat16),
        ],
    )(x, c_t, bias2)
    return out
